# Initial kernel scaffold; baseline (speedup 1.0000x reference)
import jax
import jax.numpy as jnp
from jax.experimental import pallas as pl


def kernel(x, edge_index, batch, metadata, Wrel0, Wroot0, b0, Wrel1, Wroot1, b1, Wrel2, Wroot2, b2, Wrel3, Wroot3, b3, wm, bm, fcW, fcb, fc2W, fc2b):
    raise NotImplementedError("write your pallas kernel here")


# speedup vs baseline: 4.1935x; 4.1935x over previous
import functools

import jax
import jax.numpy as jnp
from jax import lax
from jax.experimental import pallas as pl
from jax.experimental.pallas import tpu as pltpu
from jax.experimental.pallas import tpu_sc as plsc

N = 100000
G = 64
C = 32
HALF = 16
NCORES = 2
NSUB = 16
CHUNK = 128
NPAD = 100096
TROWS = NPAD // NSUB
BLK = 2000
NBLK = N // BLK

_f32 = jnp.float32


def _tc_l0_body(x_ref, wrel_ref, wroot_ref, b_ref, yrel_ref, yroot_ref):
    xb = x_ref[...]
    yrel_ref[...] = jnp.dot(xb, wrel_ref[...], preferred_element_type=_f32)
    yroot_ref[...] = jnp.dot(xb, wroot_ref[...], preferred_element_type=_f32) + b_ref[...]


def _tc_mid_body(agg_ref, yrp_ref, wrel_ref, wroot_ref, b_ref, yrel_ref, yroot_ref):
    h = jnp.concatenate([agg_ref[0], agg_ref[1]], axis=1) + yrp_ref[...]
    h = jnp.maximum(h, 0.0)
    yrel_ref[...] = jnp.dot(h, wrel_ref[...], preferred_element_type=_f32)
    yroot_ref[...] = jnp.dot(h, wroot_ref[...], preferred_element_type=_f32) + b_ref[...]


def _tc_layer0(x, wrel, wroot, b):
    return pl.pallas_call(
        _tc_l0_body,
        grid=(NBLK,),
        in_specs=[
            pl.BlockSpec((BLK, 4), lambda i: (i, 0)),
            pl.BlockSpec((4, C), lambda i: (0, 0)),
            pl.BlockSpec((4, C), lambda i: (0, 0)),
            pl.BlockSpec((1, C), lambda i: (0, 0)),
        ],
        out_specs=[
            pl.BlockSpec((BLK, C), lambda i: (i, 0)),
            pl.BlockSpec((BLK, C), lambda i: (i, 0)),
        ],
        out_shape=[
            jax.ShapeDtypeStruct((N, C), _f32),
            jax.ShapeDtypeStruct((N, C), _f32),
        ],
    )(x, wrel, wroot, b)


def _tc_mid(agg, yroot_prev, wrel, wroot, b):
    return pl.pallas_call(
        _tc_mid_body,
        grid=(NBLK,),
        in_specs=[
            pl.BlockSpec((NCORES, BLK, HALF), lambda i: (0, i, 0)),
            pl.BlockSpec((BLK, C), lambda i: (i, 0)),
            pl.BlockSpec((C, C), lambda i: (0, 0)),
            pl.BlockSpec((C, C), lambda i: (0, 0)),
            pl.BlockSpec((1, C), lambda i: (0, 0)),
        ],
        out_specs=[
            pl.BlockSpec((BLK, C), lambda i: (i, 0)),
            pl.BlockSpec((BLK, C), lambda i: (i, 0)),
        ],
        out_shape=[
            jax.ShapeDtypeStruct((N, C), _f32),
            jax.ShapeDtypeStruct((N, C), _f32),
        ],
    )(agg, yroot_prev, wrel, wroot, b)


def _pool_body(agg_ref, yrp_ref, bid_ref, meta_ref, wm_ref, bm_ref,
               fcw_ref, fcb_ref, fc2w_ref, fc2b_ref, out_ref,
               amax_ref, asum_ref, acnt_ref):
    i = pl.program_id(0)

    @pl.when(i == 0)
    def _init():
        amax_ref[...] = jnp.full((G, C), -jnp.inf, _f32)
        asum_ref[...] = jnp.zeros((G, C), _f32)
        acnt_ref[...] = jnp.zeros((G, C), _f32)

    h = jnp.concatenate([agg_ref[0], agg_ref[1]], axis=1) + yrp_ref[...]
    h = jnp.maximum(h, 0.0)
    bid = bid_ref[...]
    s_lo = jnp.min(bid)
    s_hi = jnp.max(bid)
    rows = lax.broadcasted_iota(jnp.int32, (G, C), 0)

    def seg_body(s, _):
        m = bid == s
        mf = m.astype(_f32)
        hm = jnp.where(m, h, -jnp.inf)
        smax = jnp.max(hm, axis=0, keepdims=True)
        ssum = jnp.sum(h * mf, axis=0, keepdims=True)
        scnt = jnp.sum(mf)
        upd = rows == s
        amax_ref[...] = jnp.where(upd, jnp.maximum(amax_ref[...], smax), amax_ref[...])
        asum_ref[...] = jnp.where(upd, asum_ref[...] + ssum, asum_ref[...])
        acnt_ref[...] = jnp.where(upd, acnt_ref[...] + scnt, acnt_ref[...])
        return 0

    lax.fori_loop(s_lo, s_hi + 1, seg_body, 0)

    @pl.when(i == NBLK - 1)
    def _fin():
        xmax = amax_ref[...]
        xmean = asum_ref[...] / jnp.maximum(acnt_ref[...], 1.0)
        m = jnp.maximum(meta_ref[...] * wm_ref[...] + bm_ref[...], 0.0)
        fcw = fcw_ref[...]
        z = (jnp.dot(xmax, fcw[0:C], preferred_element_type=_f32)
             + jnp.dot(xmean, fcw[C:2 * C], preferred_element_type=_f32)
             + jnp.dot(m, fcw[2 * C:2 * C + 4], preferred_element_type=_f32)
             + fcb_ref[...])
        z = jnp.maximum(z, 0.0)
        out_ref[...] = jnp.dot(z, fc2w_ref[...], preferred_element_type=_f32) + fc2b_ref[...]


def _pool_mlp(agg, yroot, bid2, meta, wm2, bm2, fcw, fcb2, fc2w, fc2b2):
    return pl.pallas_call(
        _pool_body,
        grid=(NBLK,),
        in_specs=[
            pl.BlockSpec((NCORES, BLK, HALF), lambda i: (0, i, 0)),
            pl.BlockSpec((BLK, C), lambda i: (i, 0)),
            pl.BlockSpec((BLK, 1), lambda i: (i, 0)),
            pl.BlockSpec((G, 1), lambda i: (0, 0)),
            pl.BlockSpec((1, 4), lambda i: (0, 0)),
            pl.BlockSpec((1, 4), lambda i: (0, 0)),
            pl.BlockSpec((2 * C + 4, C), lambda i: (0, 0)),
            pl.BlockSpec((1, C), lambda i: (0, 0)),
            pl.BlockSpec((C, 1), lambda i: (0, 0)),
            pl.BlockSpec((1, 1), lambda i: (0, 0)),
        ],
        out_specs=pl.BlockSpec((G, 1), lambda i: (0, 0)),
        out_shape=jax.ShapeDtypeStruct((G, 1), _f32),
        scratch_shapes=[
            pltpu.VMEM((G, C), _f32),
            pltpu.VMEM((G, C), _f32),
            pltpu.VMEM((G, C), _f32),
        ],
    )(agg, yroot, bid2, meta, wm2, bm2, fcw, fcb2, fc2w, fc2b2)


def _sc_body(nchunks, etile, t_hbm, sidx_hbm, dst_hbm, z_hbm, out_hbm,
             acc, sidx_v, didx_v, rows_v, sem):
    c = lax.axis_index("c")
    s = lax.axis_index("s")
    pltpu.sync_copy(z_hbm, acc.at[pl.ds(s * TROWS, TROWS)])
    plsc.subcore_barrier()
    base0 = s * etile

    def chunk_body(k, _):
        base = base0 + k * CHUNK
        pltpu.sync_copy(sidx_hbm.at[c, pl.ds(base, CHUNK)], sidx_v)
        pltpu.sync_copy(dst_hbm.at[pl.ds(base, CHUNK)], didx_v)
        pltpu.async_copy(t_hbm.at[sidx_v], rows_v, sem).wait()
        pltpu.sync_copy(rows_v, acc.at[didx_v], add=True)
        return 0

    lax.fori_loop(0, nchunks, chunk_body, 0)
    plsc.subcore_barrier()
    pltpu.sync_copy(acc.at[pl.ds(s * TROWS, TROWS)],
                    out_hbm.at[c, pl.ds(s * TROWS, TROWS)])


def _sc_edge_pass(ep, t2, sidx, dstp, ztile):
    etile = ep // NSUB
    nchunks = etile // CHUNK
    mesh = plsc.VectorSubcoreMesh(core_axis_name="c", subcore_axis_name="s")
    k = pl.kernel(
        functools.partial(_sc_body, nchunks, etile),
        out_type=jax.ShapeDtypeStruct((NCORES, NPAD, HALF), _f32),
        mesh=mesh,
        compiler_params=pltpu.CompilerParams(use_tc_tiling_on_sc=False),
        scratch_types=[
            pltpu.VMEM_SHARED((NPAD, HALF), _f32),
            pltpu.VMEM((CHUNK,), jnp.int32),
            pltpu.VMEM((CHUNK,), jnp.int32),
            pltpu.VMEM((CHUNK, HALF), _f32),
            pltpu.SemaphoreType.DMA,
        ],
    )
    return k(t2, sidx, dstp, ztile)


def kernel(x, edge_index, batch, metadata,
           Wrel0, Wroot0, b0, Wrel1, Wroot1, b1,
           Wrel2, Wroot2, b2, Wrel3, Wroot3, b3,
           wm, bm, fcW, fcb, fc2W, fc2b):
    e = edge_index.shape[1]
    step = NSUB * CHUNK
    ep = ((e + step - 1) // step) * step
    src = edge_index[0]
    dst = edge_index[1]
    pad = ep - e
    srcp = jnp.concatenate([src, jnp.zeros((pad,), jnp.int32)])
    dstp = jnp.concatenate([dst, jnp.full((pad,), N, jnp.int32)])
    sidx = jnp.stack([srcp * 2, srcp * 2 + 1])
    ztile = jnp.zeros((TROWS, HALF), _f32)

    layers = ((Wrel0, Wroot0, b0), (Wrel1, Wroot1, b1),
              (Wrel2, Wroot2, b2), (Wrel3, Wroot3, b3))

    yrel, yroot = _tc_layer0(x, Wrel0, Wroot0, b0.reshape(1, C))
    for wrel, wroot, b in layers[1:]:
        agg = _sc_edge_pass(ep, yrel.reshape(2 * N, HALF), sidx, dstp, ztile)
        yrel, yroot = _tc_mid(agg, yroot, wrel, wroot, b.reshape(1, C))
    agg = _sc_edge_pass(ep, yrel.reshape(2 * N, HALF), sidx, dstp, ztile)

    out = _pool_mlp(agg, yroot, batch.reshape(N, 1), metadata,
                    wm.reshape(1, 4), bm.reshape(1, 4),
                    fcW, fcb.reshape(1, C), fc2W, fc2b.reshape(1, 1))
    return out.reshape(G)
